# baseline (speedup 1.0000x reference)
import jax
import jax.numpy as jnp
from jax import lax
from jax.experimental import pallas as pl
from jax.experimental.pallas import tpu as pltpu
from jax.experimental.pallas import tpu_sc as plsc

N = 10000
E = 320000
D = 128
NC = 2
NS = 16
NW = NC * NS
C = 32
NCH = E // C
NCH_BASE = NCH // NW
NCH_REM = NCH % NW
NP = 10240
RPT = NP // NS
SN = NP // 10


def _leaky(x, s):
  return jnp.maximum(x, s * x)


def _tc_pre_body(x_ref, g_ref, b_ref, wlt_ref, bl_ref, wrt_ref, br_ref,
                 xl_ref, xr_ref):
  x = x_ref[...]
  mean = jnp.mean(x, axis=0, keepdims=True)
  xc = x - mean
  var = jnp.mean(xc * xc, axis=0, keepdims=True)
  h = xc * lax.rsqrt(var + 1e-5) * g_ref[...] + b_ref[...]
  h = _leaky(h, 0.1)
  xl_ref[...] = jnp.dot(h, wlt_ref[...], preferred_element_type=jnp.float32) + bl_ref[...]
  xr_ref[...] = jnp.dot(h, wrt_ref[...], preferred_element_type=jnp.float32) + br_ref[...]


def _tc_pre(x, g, b, wlt, bl, wrt, br):
  return pl.pallas_call(
      _tc_pre_body,
      out_shape=(jax.ShapeDtypeStruct((N, D), jnp.float32),
                 jax.ShapeDtypeStruct((N, D), jnp.float32)),
  )(x, g, b, wlt, bl, wrt, br)


def _gat_out(acc_ref, den_ref, bias_ref):
  acc3 = acc_ref[...].reshape(NC, NP // D, D, D)
  acc3 = acc3[0] + acc3[1]
  den = den_ref[0] + den_ref[1]
  out3 = acc3 / (den[..., None] + 1e-16)
  return out3.reshape(NP, D)[:N] + bias_ref[...]


def _tc_mid_body(acc_ref, den_ref, bias_ref, g_ref, b_ref,
                 wlt_ref, bl_ref, wrt_ref, br_ref, xl_ref, xr_ref):
  out = _gat_out(acc_ref, den_ref, bias_ref)
  h = _leaky(out, 0.1)
  mean = jnp.mean(h, axis=0, keepdims=True)
  hc = h - mean
  var = jnp.mean(hc * hc, axis=0, keepdims=True)
  h = hc * lax.rsqrt(var + 1e-5) * g_ref[...] + b_ref[...]
  h = _leaky(h, 0.1)
  xl_ref[...] = jnp.dot(h, wlt_ref[...], preferred_element_type=jnp.float32) + bl_ref[...]
  xr_ref[...] = jnp.dot(h, wrt_ref[...], preferred_element_type=jnp.float32) + br_ref[...]


def _tc_mid(acc, den, bias, g, b, wlt, bl, wrt, br):
  return pl.pallas_call(
      _tc_mid_body,
      out_shape=(jax.ShapeDtypeStruct((N, D), jnp.float32),
                 jax.ShapeDtypeStruct((N, D), jnp.float32)),
  )(acc, den, bias, g, b, wlt, bl, wrt, br)


def _tc_post_body(acc_ref, den_ref, bias_ref, out_ref):
  out = _gat_out(acc_ref, den_ref, bias_ref)
  out_ref[...] = _leaky(out, 0.1)


def _tc_post(acc, den, bias):
  return pl.pallas_call(
      _tc_post_body,
      out_shape=jax.ShapeDtypeStruct((N, D), jnp.float32),
  )(acc, den, bias)


def _sc_edge_body(xl_hbm, xr_hbm, src_hbm, dst_hbm, ea_hbm, we_hbm, att_hbm,
                  acc_out, den_out,
                  acc_sh, den_sh,
                  sidx, didx, eav, xlv, xrv, outv, denl, dbuf, dacc,
                  wev, attv, sem):
  cid = lax.axis_index("c")
  sid = lax.axis_index("s")
  wid = sid * NC + cid

  pltpu.sync_copy(we_hbm, wev)
  pltpu.sync_copy(att_hbm, attv)

  z16 = jnp.zeros((16,), jnp.float32)

  def zrow(i, carry):
    for j in range(8):
      outv[i, pl.ds(j * 16, 16)] = z16
    return carry

  lax.fori_loop(0, C, zrow, 0)

  def zden(i, carry):
    denl[pl.ds(i * 16, 16)] = z16
    return carry

  lax.fori_loop(0, NP // 16, zden, 0)
  rbase = sid * RPT
  for k in range(RPT // C):
    pltpu.sync_copy(outv, acc_sh.at[pl.ds(rbase + k * C, C)])
  plsc.subcore_barrier()

  def chunk(ci, carry):
    base = pl.multiple_of((wid + ci * NW) * C, 8)
    pltpu.sync_copy(src_hbm.at[pl.ds(base, C)], sidx)
    pltpu.sync_copy(dst_hbm.at[pl.ds(base, C)], didx)
    pltpu.sync_copy(ea_hbm.at[pl.ds(base * 4, C * 4)], eav)
    cp1 = pltpu.async_copy(xl_hbm.at[sidx], xlv, sem)
    cp2 = pltpu.async_copy(xr_hbm.at[didx], xrv, sem)
    cp1.wait()
    cp2.wait()

    riota = lax.iota(jnp.int32, 16)
    dnums = lax.GatherDimensionNumbers(
        offset_dims=(), collapsed_slice_dims=(0,), start_index_map=(0,))

    def group(g, gcarry):
      dv16 = didx[pl.ds(g * 16, 16)]
      avec = None
      for i in range(16):
        e = g * 16 + i
        if i % 4 == 0:
          avec = eav[pl.ds(g * 64 + (i // 4) * 16, 16)]
        ea0 = avec[4 * (i % 4)]
        ea1 = avec[4 * (i % 4) + 1]
        ea2 = avec[4 * (i % 4) + 2]
        ea3 = avec[4 * (i % 4) + 3]
        t = None
        xls = []
        for j in range(8):
          sl = pl.ds(j * 16, 16)
          xlj = xlv[e, sl]
          xls.append(xlj)
          v = xlj + xrv[e, sl]
          v = (v + ea0 * wev[pl.ds(j * 16, 16)]
               + ea1 * wev[pl.ds(128 + j * 16, 16)]
               + ea2 * wev[pl.ds(256 + j * 16, 16)]
               + ea3 * wev[pl.ds(384 + j * 16, 16)])
          m = jnp.maximum(v, 0.2 * v)
          c = m * attv[pl.ds(j * 16, 16)]
          t = c if t is None else t + c
        for k in (8, 4, 2, 1):
          idx = riota ^ k
          t = t + lax.gather(t, idx[:, None], dnums, slice_sizes=(1,),
                             mode=lax.GatherScatterMode.PROMISE_IN_BOUNDS)
        ex = jnp.exp(t)
        for j in range(8):
          outv[e, pl.ds(j * 16, 16)] = ex * xls[j]
        d_i = dv16[i]
        dwin = lax.shift_left(lax.shift_right_logical(d_i, 4), 4)
        exm = jnp.where(riota == lax.bitwise_and(d_i, 15), ex, z16)
        denl[pl.ds(dwin, 16)] = denl[pl.ds(dwin, 16)] + exm
      return gcarry

    lax.fori_loop(0, C // 16, group, 0)
    pltpu.sync_copy(outv, acc_sh.at[didx], add=True)
    return carry

  nchunks = NCH_BASE + jnp.where(wid < NCH_REM, 1, 0)
  lax.fori_loop(0, nchunks, chunk, 0)

  pltpu.sync_copy(denl, den_sh.at[pl.ds(sid * NP, NP)])
  plsc.subcore_barrier()

  for k in range(5):
    pltpu.sync_copy(acc_sh.at[pl.ds(rbase + k * 128, 128)],
                    acc_out.at[cid, pl.ds(rbase + k * 128, 128)])

  @pl.when(sid < 10)
  def _():
    nbase = sid * SN
    pltpu.sync_copy(den_sh.at[pl.ds(nbase, SN)], dacc)

    def tsum(t, carry):
      pltpu.sync_copy(den_sh.at[pl.ds(t * NP + nbase, SN)], dbuf)

      def win(m, mcarry):
        sl = pl.ds(m * 16, 16)
        dacc[sl] = dacc[sl] + dbuf[sl]
        return mcarry

      lax.fori_loop(0, SN // 16, win, 0)
      return carry

    lax.fori_loop(1, NS, tsum, 0)
    pltpu.sync_copy(dacc, den_out.at[pl.ds(cid * NP + nbase, SN)])


_sc_edge = pl.kernel(
    _sc_edge_body,
    out_type=(jax.ShapeDtypeStruct((NC, NP, D), jnp.float32),
              jax.ShapeDtypeStruct((NC * NP,), jnp.float32)),
    mesh=plsc.VectorSubcoreMesh(core_axis_name="c", subcore_axis_name="s",
                                num_cores=NC),
    scratch_types=[
        pltpu.VMEM_SHARED((NP, D), jnp.float32),
        pltpu.VMEM_SHARED((NS * NP,), jnp.float32),
        pltpu.VMEM((C,), jnp.int32),
        pltpu.VMEM((C,), jnp.int32),
        pltpu.VMEM((C * 4,), jnp.float32),
        pltpu.VMEM((C, D), jnp.float32),
        pltpu.VMEM((C, D), jnp.float32),
        pltpu.VMEM((C, D), jnp.float32),
        pltpu.VMEM((NP,), jnp.float32),
        pltpu.VMEM((SN,), jnp.float32),
        pltpu.VMEM((SN,), jnp.float32),
        pltpu.VMEM((512,), jnp.float32),
        pltpu.VMEM((D,), jnp.float32),
        pltpu.SemaphoreType.DMA,
    ],
)


def kernel(x, edge_index, edge_attr, bn1_g, bn1_b, Wl1, bl1, Wr1, br1, We1,
           att1, bias1, bn2_g, bn2_b, Wl2, bl2, Wr2, br2, We2, att2, bias2):
  src = edge_index[0].astype(jnp.int32)
  dst = edge_index[1].astype(jnp.int32)
  eaf = edge_attr.reshape(E * 4)

  def row(v):
    return v.reshape(1, D).astype(jnp.float32)

  we1r = We1.T.reshape(512)
  we2r = We2.T.reshape(512)
  att1r = att1.reshape(D)
  att2r = att2.reshape(D)

  xl1, xr1 = _tc_pre(x, row(bn1_g), row(bn1_b), Wl1.T, row(bl1), Wr1.T, row(br1))
  acc1, den1 = _sc_edge(xl1, xr1, src, dst, eaf, we1r, att1r)
  den1 = den1.reshape(NC, NP // D, D)
  xl2, xr2 = _tc_mid(acc1, den1, row(bias1), row(bn2_g), row(bn2_b),
                     Wl2.T, row(bl2), Wr2.T, row(br2))
  acc2, den2 = _sc_edge(xl2, xr2, src, dst, eaf, we2r, att2r)
  den2 = den2.reshape(NC, NP // D, D)
  out = _tc_post(acc2, den2, row(bias2))
  return (out, edge_index)


# speedup vs baseline: 2.5400x; 2.5400x over previous
import jax
import jax.numpy as jnp
from jax import lax
from jax.experimental import pallas as pl
from jax.experimental.pallas import tpu as pltpu
from jax.experimental.pallas import tpu_sc as plsc

N = 10000
E = 320000
D = 128
NC = 2
NS = 16
NW = NC * NS
C = 64
NCH = E // C
NCH_BASE = NCH // NW
NCH_REM = NCH % NW
NP = 10240
RPT = NP // NS
SN = NP // 10


def _leaky(x, s):
  return jnp.maximum(x, s * x)


def _tc_pre_body(x_ref, g_ref, b_ref, wlt_ref, bl_ref, wrt_ref, br_ref,
                 xl_ref, xr_ref):
  x = x_ref[...]
  mean = jnp.mean(x, axis=0, keepdims=True)
  xc = x - mean
  var = jnp.mean(xc * xc, axis=0, keepdims=True)
  h = xc * lax.rsqrt(var + 1e-5) * g_ref[...] + b_ref[...]
  h = _leaky(h, 0.1)
  xl_ref[...] = jnp.dot(h, wlt_ref[...], preferred_element_type=jnp.float32) + bl_ref[...]
  xr_ref[...] = jnp.dot(h, wrt_ref[...], preferred_element_type=jnp.float32) + br_ref[...]


def _tc_pre(x, g, b, wlt, bl, wrt, br):
  return pl.pallas_call(
      _tc_pre_body,
      out_shape=(jax.ShapeDtypeStruct((N, D), jnp.float32),
                 jax.ShapeDtypeStruct((N, D), jnp.float32)),
  )(x, g, b, wlt, bl, wrt, br)


def _gat_out(acc_ref, den_ref, bias_ref):
  acc3 = acc_ref[...].reshape(NC, NP // D, D, D)
  acc3 = acc3[0] + acc3[1]
  den = den_ref[0] + den_ref[1]
  out3 = acc3 / (den[..., None] + 1e-16)
  return out3.reshape(NP, D)[:N] + bias_ref[...]


def _tc_mid_body(acc_ref, den_ref, bias_ref, g_ref, b_ref,
                 wlt_ref, bl_ref, wrt_ref, br_ref, xl_ref, xr_ref):
  out = _gat_out(acc_ref, den_ref, bias_ref)
  h = _leaky(out, 0.1)
  mean = jnp.mean(h, axis=0, keepdims=True)
  hc = h - mean
  var = jnp.mean(hc * hc, axis=0, keepdims=True)
  h = hc * lax.rsqrt(var + 1e-5) * g_ref[...] + b_ref[...]
  h = _leaky(h, 0.1)
  xl_ref[...] = jnp.dot(h, wlt_ref[...], preferred_element_type=jnp.float32) + bl_ref[...]
  xr_ref[...] = jnp.dot(h, wrt_ref[...], preferred_element_type=jnp.float32) + br_ref[...]


def _tc_mid(acc, den, bias, g, b, wlt, bl, wrt, br):
  return pl.pallas_call(
      _tc_mid_body,
      out_shape=(jax.ShapeDtypeStruct((N, D), jnp.float32),
                 jax.ShapeDtypeStruct((N, D), jnp.float32)),
  )(acc, den, bias, g, b, wlt, bl, wrt, br)


def _tc_post_body(acc_ref, den_ref, bias_ref, out_ref):
  out = _gat_out(acc_ref, den_ref, bias_ref)
  out_ref[...] = _leaky(out, 0.1)


def _tc_post(acc, den, bias):
  return pl.pallas_call(
      _tc_post_body,
      out_shape=jax.ShapeDtypeStruct((N, D), jnp.float32),
  )(acc, den, bias)


def _sc_edge_body(xl_hbm, xr_hbm, src_hbm, dst_hbm, ea_hbm, we_hbm, att_hbm,
                  acc_out, den_out,
                  acc_sh, den_sh,
                  sidx, didx, eav, xlv, xrv, outv, denl, dbuf, dacc,
                  wev, attv, sem):
  cid = lax.axis_index("c")
  sid = lax.axis_index("s")
  wid = sid * NC + cid

  pltpu.sync_copy(we_hbm, wev)
  pltpu.sync_copy(att_hbm, attv)

  z16 = jnp.zeros((16,), jnp.float32)

  def zrow(i, carry):
    for j in range(8):
      outv[i, pl.ds(j * 16, 16)] = z16
    return carry

  lax.fori_loop(0, C, zrow, 0)

  def zden(i, carry):
    denl[pl.ds(i * 16, 16)] = z16
    return carry

  lax.fori_loop(0, NP // 16, zden, 0)
  rbase = sid * RPT
  for k in range(RPT // C):
    pltpu.sync_copy(outv, acc_sh.at[pl.ds(rbase + k * C, C)])
  plsc.subcore_barrier()

  def chunk(ci, carry):
    base = pl.multiple_of((wid + ci * NW) * C, 8)
    pltpu.sync_copy(src_hbm.at[pl.ds(base, C)], sidx)
    pltpu.sync_copy(dst_hbm.at[pl.ds(base, C)], didx)
    pltpu.sync_copy(ea_hbm.at[pl.ds(base * 4, C * 4)], eav)
    cp1 = pltpu.async_copy(xl_hbm.at[sidx], xlv, sem)
    cp2 = pltpu.async_copy(xr_hbm.at[didx], xrv, sem)
    cp1.wait()
    cp2.wait()

    riota = lax.iota(jnp.int32, 16)
    dnums = lax.GatherDimensionNumbers(
        offset_dims=(), collapsed_slice_dims=(0,), start_index_map=(0,))

    def group(g, gcarry):
      dv16 = didx[pl.ds(g * 16, 16)]
      ts = []
      avec = None
      for i in range(16):
        e = g * 16 + i
        if i % 4 == 0:
          avec = eav[pl.ds(g * 64 + (i // 4) * 16, 16)]
        ea0 = avec[4 * (i % 4)]
        ea1 = avec[4 * (i % 4) + 1]
        ea2 = avec[4 * (i % 4) + 2]
        ea3 = avec[4 * (i % 4) + 3]
        t = None
        for j in range(8):
          sl = pl.ds(j * 16, 16)
          v = xlv[e, sl] + xrv[e, sl]
          v = (v + ea0 * wev[pl.ds(j * 16, 16)]
               + ea1 * wev[pl.ds(128 + j * 16, 16)]
               + ea2 * wev[pl.ds(256 + j * 16, 16)]
               + ea3 * wev[pl.ds(384 + j * 16, 16)])
          m = jnp.maximum(v, 0.2 * v)
          c = m * attv[pl.ds(j * 16, 16)]
          t = c if t is None else t + c
        ts.append(t)
      vecs = ts
      for k in (1, 2, 4, 8):
        mask = lax.bitwise_and(riota, k) == 0
        perm = (riota ^ k)[:, None]
        nxt = []
        for p in range(len(vecs) // 2):
          x, y = vecs[2 * p], vecs[2 * p + 1]
          z1 = jnp.where(mask, x, y)
          z2 = jnp.where(mask, y, x)
          nxt.append(z1 + lax.gather(z2, perm, dnums, slice_sizes=(1,),
                                     mode=lax.GatherScatterMode.PROMISE_IN_BOUNDS))
        vecs = nxt
      ex16 = jnp.exp(vecs[0])
      spls = []
      for i in range(16):
        e = g * 16 + i
        spl = jnp.broadcast_to(ex16[i], (16,))
        spls.append(spl)
        for j in range(8):
          sl = pl.ds(j * 16, 16)
          outv[e, sl] = spl * xlv[e, sl]
      for i in range(16):
        d_i = dv16[i]
        dwin = lax.shift_left(lax.shift_right_logical(d_i, 4), 4)
        exm = jnp.where(riota == lax.bitwise_and(d_i, 15), spls[i], z16)
        denl[pl.ds(dwin, 16)] = denl[pl.ds(dwin, 16)] + exm
      return gcarry

    lax.fori_loop(0, C // 16, group, 0)
    pltpu.sync_copy(outv, acc_sh.at[didx], add=True)
    return carry

  nchunks = NCH_BASE + jnp.where(wid < NCH_REM, 1, 0)
  lax.fori_loop(0, nchunks, chunk, 0)

  pltpu.sync_copy(denl, den_sh.at[pl.ds(sid * NP, NP)])
  plsc.subcore_barrier()

  for k in range(5):
    pltpu.sync_copy(acc_sh.at[pl.ds(rbase + k * 128, 128)],
                    acc_out.at[cid, pl.ds(rbase + k * 128, 128)])

  @pl.when(sid < 10)
  def _():
    nbase = sid * SN
    pltpu.sync_copy(den_sh.at[pl.ds(nbase, SN)], dacc)

    def tsum(t, carry):
      pltpu.sync_copy(den_sh.at[pl.ds(t * NP + nbase, SN)], dbuf)

      def win(m, mcarry):
        sl = pl.ds(m * 16, 16)
        dacc[sl] = dacc[sl] + dbuf[sl]
        return mcarry

      lax.fori_loop(0, SN // 16, win, 0)
      return carry

    lax.fori_loop(1, NS, tsum, 0)
    pltpu.sync_copy(dacc, den_out.at[pl.ds(cid * NP + nbase, SN)])


_sc_edge = pl.kernel(
    _sc_edge_body,
    out_type=(jax.ShapeDtypeStruct((NC, NP, D), jnp.float32),
              jax.ShapeDtypeStruct((NC * NP,), jnp.float32)),
    mesh=plsc.VectorSubcoreMesh(core_axis_name="c", subcore_axis_name="s",
                                num_cores=NC),
    scratch_types=[
        pltpu.VMEM_SHARED((NP, D), jnp.float32),
        pltpu.VMEM_SHARED((NS * NP,), jnp.float32),
        pltpu.VMEM((C,), jnp.int32),
        pltpu.VMEM((C,), jnp.int32),
        pltpu.VMEM((C * 4,), jnp.float32),
        pltpu.VMEM((C, D), jnp.float32),
        pltpu.VMEM((C, D), jnp.float32),
        pltpu.VMEM((C, D), jnp.float32),
        pltpu.VMEM((NP,), jnp.float32),
        pltpu.VMEM((SN,), jnp.float32),
        pltpu.VMEM((SN,), jnp.float32),
        pltpu.VMEM((512,), jnp.float32),
        pltpu.VMEM((D,), jnp.float32),
        pltpu.SemaphoreType.DMA,
    ],
)


def kernel(x, edge_index, edge_attr, bn1_g, bn1_b, Wl1, bl1, Wr1, br1, We1,
           att1, bias1, bn2_g, bn2_b, Wl2, bl2, Wr2, br2, We2, att2, bias2):
  src = edge_index[0].astype(jnp.int32)
  dst = edge_index[1].astype(jnp.int32)
  eaf = edge_attr.reshape(E * 4)

  def row(v):
    return v.reshape(1, D).astype(jnp.float32)

  we1r = We1.T.reshape(512)
  we2r = We2.T.reshape(512)
  att1r = att1.reshape(D)
  att2r = att2.reshape(D)

  xl1, xr1 = _tc_pre(x, row(bn1_g), row(bn1_b), Wl1.T, row(bl1), Wr1.T, row(br1))
  acc1, den1 = _sc_edge(xl1, xr1, src, dst, eaf, we1r, att1r)
  den1 = den1.reshape(NC, NP // D, D)
  xl2, xr2 = _tc_mid(acc1, den1, row(bias1), row(bn2_g), row(bn2_b),
                     Wl2.T, row(bl2), Wr2.T, row(br2))
  acc2, den2 = _sc_edge(xl2, xr2, src, dst, eaf, we2r, att2r)
  den2 = den2.reshape(NC, NP // D, D)
  out = _tc_post(acc2, den2, row(bias2))
  return (out, edge_index)
